# baseline (speedup 1.0000x reference)
import functools
import jax
import jax.numpy as jnp
from jax.experimental import pallas as pl
from jax.experimental.pallas import tpu as pltpu
from jax.experimental.pallas import tpu_sc as plsc

_BN = 4
_LANES = 16


def kernel(node_input, affine_weight, expand_index):
    n, m, c = node_input.shape
    num_l = affine_weight.shape[1]
    aw2 = affine_weight.reshape(num_l, c)
    ei = expand_index.astype(jnp.int32)
    kc = c // _LANES

    mesh = plsc.VectorSubcoreMesh(core_axis_name="core", subcore_axis_name="subcore")

    @functools.partial(
        pl.kernel,
        out_type=jax.ShapeDtypeStruct((n, m, c), jnp.float32),
        mesh=mesh,
        scratch_types=[
            pltpu.VMEM((m,), jnp.int32),
            pltpu.VMEM((m, c), jnp.float32),
            pltpu.SemaphoreType.DMA,
        ],
    )
    def _sc(x_hbm, aw_hbm, ei_hbm, o_hbm, ei_v, w_v, sem):
        pltpu.async_copy(ei_hbm, ei_v, sem).wait()
        pltpu.async_copy(aw_hbm.at[ei_v], w_v, sem).wait()

        def body(in_v, out_v):
            for nn in range(_BN):
                for mm in range(m):
                    for k in range(kc):
                        sl = pl.ds(k * _LANES, _LANES)
                        out_v[nn, mm, sl] = in_v[nn, mm, sl] * w_v[mm, sl]

        pltpu.emit_pipeline(
            body,
            grid=(n // _BN,),
            in_specs=[pl.BlockSpec((_BN, m, c), lambda i: (i, 0, 0))],
            out_specs=[pl.BlockSpec((_BN, m, c), lambda i: (i, 0, 0))],
            core_axis_name=("core", "subcore"),
            dimension_semantics=(pltpu.PARALLEL,),
        )(x_hbm, o_hbm)

    return _sc(node_input, aw2, ei)


# speedup vs baseline: 8.3448x; 8.3448x over previous
import jax
import jax.numpy as jnp
from jax.experimental import pallas as pl
from jax.experimental.pallas import tpu as pltpu

_NSPLIT = 2


def _scale_body(ei_ref, aw_ref, x_ref, o_ref, w_ref):
    m = ei_ref.shape[0]
    num_l = aw_ref.shape[0]

    @pl.when((pl.program_id(0) == 0) & (pl.program_id(1) == 0))
    def _():
        ei = ei_ref[...]
        onehot = (ei == jax.lax.broadcasted_iota(jnp.int32, (m, num_l), 1))
        w_ref[...] = jax.lax.dot_general(
            onehot.astype(jnp.float32), aw_ref[...],
            (((1,), (0,)), ((), ())),
            preferred_element_type=jnp.float32)

    i = pl.program_id(0)
    o_ref[...] = x_ref[...] * w_ref[pl.ds(i, 1), :][None]


def kernel(node_input, affine_weight, expand_index):
    n, m, c = node_input.shape
    x_t = jnp.transpose(node_input, (1, 0, 2))
    aw = affine_weight.reshape(affine_weight.shape[-2], c)
    ei = expand_index.astype(jnp.int32).reshape(m, 1)

    bn = n // _NSPLIT
    out_t = pl.pallas_call(
        _scale_body,
        grid=(m, _NSPLIT),
        in_specs=[
            pl.BlockSpec((m, 1), lambda i, j: (0, 0)),
            pl.BlockSpec(aw.shape, lambda i, j: (0, 0)),
            pl.BlockSpec((1, bn, c), lambda i, j: (i, j, 0)),
        ],
        out_specs=pl.BlockSpec((1, bn, c), lambda i, j: (i, j, 0)),
        out_shape=jax.ShapeDtypeStruct((m, n, c), jnp.float32),
        scratch_shapes=[pltpu.VMEM((m, c), jnp.float32)],
    )(ei, aw, x_t)
    return jnp.transpose(out_t, (1, 0, 2))
